# baseline (speedup 1.0000x reference)
import functools

import jax
import jax.numpy as jnp
from jax import lax
from jax.experimental import pallas as pl
from jax.experimental.pallas import tpu as pltpu
from jax.experimental.pallas import tpu_sc as plsc

VOCAB = 1000000
EMBED_DIM = 64
BATCH = 16384
HIST_LEN = 20

B = BATCH * HIST_LEN
NC, NS = 2, 16
NW = NC * NS
B_PER_W = B // NW
CHUNK = 512
NCHUNK = B_PER_W // CHUNK

_mesh = plsc.VectorSubcoreMesh(core_axis_name="c", subcore_axis_name="s")


@functools.partial(
    pl.kernel,
    out_type=jax.ShapeDtypeStruct((B, EMBED_DIM), jnp.float32),
    mesh=_mesh,
    scratch_types=[
        pltpu.VMEM((CHUNK,), jnp.int32),
        pltpu.VMEM((CHUNK, EMBED_DIM), jnp.float32),
        pltpu.SemaphoreType.DMA,
    ],
    compiler_params=pltpu.CompilerParams(use_tc_tiling_on_sc=False),
)
def _gather_kernel(table_hbm, idx_hbm, out_hbm, idx_v, rows_v, gsem):
    wid = lax.axis_index("s") * NC + lax.axis_index("c")
    base = wid * B_PER_W

    @pl.loop(0, NCHUNK)
    def _(i):
        off = base + i * CHUNK
        pltpu.sync_copy(idx_hbm.at[pl.ds(off, CHUNK)], idx_v)
        pltpu.async_copy(table_hbm.at[idx_v], rows_v, gsem).wait()
        pltpu.sync_copy(rows_v, out_hbm.at[pl.ds(off, CHUNK)])


def kernel(weight, words):
    idx = words.astype(jnp.int32).reshape(-1)
    out = _gather_kernel(weight, idx)
    return out.reshape(BATCH, HIST_LEN, EMBED_DIM)


# speedup vs baseline: 1.0210x; 1.0210x over previous
import functools

import jax
import jax.numpy as jnp
from jax import lax
from jax.experimental import pallas as pl
from jax.experimental.pallas import tpu as pltpu
from jax.experimental.pallas import tpu_sc as plsc

VOCAB = 1000000
EMBED_DIM = 64
BATCH = 16384
HIST_LEN = 20

B = BATCH * HIST_LEN
NC, NS = 2, 16
NW = NC * NS
B_PER_W = B // NW
CHUNK = 512
NCHUNK = B_PER_W // CHUNK
NBUF = 2
NGRP = (NCHUNK - NBUF) // NBUF

_mesh = plsc.VectorSubcoreMesh(core_axis_name="c", subcore_axis_name="s")


@functools.partial(
    pl.kernel,
    out_type=jax.ShapeDtypeStruct((B, EMBED_DIM), jnp.float32),
    mesh=_mesh,
    scratch_types=[
        pltpu.VMEM((B_PER_W,), jnp.int32),
        pltpu.VMEM((NBUF, CHUNK, EMBED_DIM), jnp.float32),
        [pltpu.SemaphoreType.DMA] * NBUF,
        [pltpu.SemaphoreType.DMA] * NBUF,
    ],
    compiler_params=pltpu.CompilerParams(use_tc_tiling_on_sc=False),
)
def _gather_kernel(table_hbm, idx_hbm, out_hbm, idx_v, rows_v, gsems, osems):
    wid = lax.axis_index("s") * NC + lax.axis_index("c")
    base = wid * B_PER_W

    pltpu.sync_copy(idx_hbm.at[pl.ds(base, B_PER_W)], idx_v)

    def start_gather(i, b):
        pltpu.async_copy(
            table_hbm.at[idx_v.at[pl.ds(i * CHUNK, CHUNK)]],
            rows_v.at[b], gsems[b])

    def start_store(i, b):
        pltpu.async_copy(
            rows_v.at[b], out_hbm.at[pl.ds(base + i * CHUNK, CHUNK)],
            osems[b])

    def wait_gather(b):
        pltpu.make_async_copy(
            out_hbm.at[pl.ds(base, CHUNK)], rows_v.at[b], gsems[b]).wait()

    def wait_store(b):
        pltpu.make_async_copy(
            rows_v.at[b], out_hbm.at[pl.ds(base, CHUNK)], osems[b]).wait()

    for b in range(NBUF):
        start_gather(b, b)

    @pl.loop(0, NGRP)
    def _(g):
        for b in range(NBUF):
            i = g * NBUF + b
            wait_gather(b)
            start_store(i, b)
            wait_store(b)
            start_gather(i + NBUF, b)

    for b in range(NBUF):
        i = NCHUNK - NBUF + b
        wait_gather(b)
        start_store(i, b)
    for b in range(NBUF):
        wait_store(b)


def kernel(weight, words):
    idx = words.astype(jnp.int32).reshape(-1)
    out = _gather_kernel(weight, idx)
    return out.reshape(BATCH, HIST_LEN, EMBED_DIM)


# speedup vs baseline: 1.0607x; 1.0390x over previous
import functools

import jax
import jax.numpy as jnp
from jax import lax
from jax.experimental import pallas as pl
from jax.experimental.pallas import tpu as pltpu
from jax.experimental.pallas import tpu_sc as plsc

VOCAB = 1000000
EMBED_DIM = 64
PADDED_DIM = 128
BATCH = 16384
HIST_LEN = 20

B = BATCH * HIST_LEN
NC, NS = 2, 16
NW = NC * NS
B_PER_W = B // NW
NB = 16
CHUNK = NB * HIST_LEN
NCHUNK = B_PER_W // CHUNK
NBUF = 2
NGRP = (NCHUNK - NBUF) // NBUF

_mesh = plsc.VectorSubcoreMesh(core_axis_name="c", subcore_axis_name="s")


@functools.partial(
    pl.kernel,
    out_type=jax.ShapeDtypeStruct((BATCH, HIST_LEN, EMBED_DIM), jnp.float32),
    mesh=_mesh,
    scratch_types=[
        pltpu.VMEM((B_PER_W,), jnp.int32),
        pltpu.VMEM((NBUF, CHUNK, PADDED_DIM), jnp.float32),
        [pltpu.SemaphoreType.DMA] * NBUF,
        [pltpu.SemaphoreType.DMA] * NBUF,
    ],
    compiler_params=pltpu.CompilerParams(use_tc_tiling_on_sc=False),
)
def _gather_kernel(table_hbm, idx_hbm, out_hbm, idx_v, rows_v, gsems, osems):
    wid = lax.axis_index("s") * NC + lax.axis_index("c")
    base = wid * B_PER_W
    bbase = wid * (B_PER_W // HIST_LEN)

    pltpu.sync_copy(idx_hbm.at[pl.ds(base, B_PER_W)], idx_v)

    def start_gather(i, b):
        pltpu.async_copy(
            table_hbm.at[idx_v.at[pl.ds(i * CHUNK, CHUNK)]],
            rows_v.at[b], gsems[b])

    def start_store(i, b):
        for k in range(NB):
            pltpu.async_copy(
                rows_v.at[b, pl.ds(k * HIST_LEN, HIST_LEN), pl.ds(0, EMBED_DIM)],
                out_hbm.at[bbase + i * NB + k],
                osems[b])

    def wait_gather(b):
        pltpu.make_async_copy(
            table_hbm.at[pl.ds(0, CHUNK)], rows_v.at[b],
            gsems[b]).wait()

    def wait_store(b):
        for k in range(NB):
            pltpu.make_async_copy(
                rows_v.at[b, pl.ds(k * HIST_LEN, HIST_LEN), pl.ds(0, EMBED_DIM)],
                out_hbm.at[bbase + k],
                osems[b]).wait()

    for b in range(NBUF):
        start_gather(b, b)

    @pl.loop(0, NGRP)
    def _(g):
        for b in range(NBUF):
            i = g * NBUF + b
            wait_gather(b)
            start_store(i, b)
            wait_store(b)
            start_gather(i + NBUF, b)

    for b in range(NBUF):
        i = NCHUNK - NBUF + b
        wait_gather(b)
        start_store(i, b)
    for b in range(NBUF):
        wait_store(b)


def kernel(weight, words):
    w128 = jnp.pad(weight, ((0, 0), (0, PADDED_DIM - EMBED_DIM)))
    idx = words.astype(jnp.int32).reshape(-1)
    return _gather_kernel(w128, idx)
